# baseline (speedup 1.0000x reference)
import functools

import jax
import jax.numpy as jnp
from jax import lax
from jax.experimental import pallas as pl
from jax.experimental.pallas import tpu as pltpu
from jax.experimental.pallas import tpu_sc as plsc

VOCAB = 1000000
EMBED = 64
BATCH = 4096
WINDOW = 200

TOTAL = 2 * BATCH * WINDOW
NUM_CORES = 2
NUM_SUBCORES = 16
NW = NUM_CORES * NUM_SUBCORES
PER_W = TOTAL // NW
CHUNK = 512
NCHUNK = PER_W // CHUNK

_mesh = plsc.VectorSubcoreMesh(
    core_axis_name="c", subcore_axis_name="s",
    num_cores=NUM_CORES, num_subcores=NUM_SUBCORES,
)


@functools.partial(
    pl.kernel,
    out_type=jax.ShapeDtypeStruct((TOTAL, EMBED), jnp.float32),
    mesh=_mesh,
    scratch_types=[
        pltpu.VMEM((CHUNK,), jnp.int32),
        pltpu.VMEM((CHUNK, EMBED), jnp.float32),
        pltpu.SemaphoreType.DMA,
    ],
    compiler_params=pltpu.CompilerParams(use_tc_tiling_on_sc=False),
)
def _gather_all(table_hbm, idx_hbm, out_hbm, idx_v, rows_v, sem):
    wid = lax.axis_index("s") * NUM_CORES + lax.axis_index("c")
    base = wid * PER_W

    def body(i, carry):
        off = base + i * CHUNK
        pltpu.sync_copy(idx_hbm.at[pl.ds(off, CHUNK)], idx_v)
        pltpu.async_copy(table_hbm.at[idx_v], rows_v, sem).wait()
        pltpu.sync_copy(rows_v, out_hbm.at[pl.ds(off, CHUNK)])
        return carry

    lax.fori_loop(0, NCHUNK, body, 0)


def kernel(inputs, labels, E):
    idx = jnp.concatenate(
        [inputs.reshape(-1), labels.reshape(-1)]).astype(jnp.int32)
    out = _gather_all(E, idx)
    return out.reshape(2, BATCH, WINDOW, EMBED)


# speedup vs baseline: 1.0623x; 1.0623x over previous
import functools

import jax
import jax.numpy as jnp
from jax import lax
from jax.experimental import pallas as pl
from jax.experimental.pallas import tpu as pltpu
from jax.experimental.pallas import tpu_sc as plsc

VOCAB = 1000000
EMBED = 64
BATCH = 4096
WINDOW = 200

TOTAL = 2 * BATCH * WINDOW
NUM_CORES = 2
NUM_SUBCORES = 16
NW = NUM_CORES * NUM_SUBCORES
PER_W = TOTAL // NW
CHUNK = 256
NBUF = 4
GROUP = NBUF * CHUNK
NGROUP = PER_W // GROUP

_mesh = plsc.VectorSubcoreMesh(
    core_axis_name="c", subcore_axis_name="s",
    num_cores=NUM_CORES, num_subcores=NUM_SUBCORES,
)


@functools.partial(
    pl.kernel,
    out_type=jax.ShapeDtypeStruct((TOTAL, EMBED), jnp.float32),
    mesh=_mesh,
    scratch_types=[
        pltpu.VMEM((PER_W,), jnp.int32),
        pltpu.VMEM((NBUF, CHUNK, EMBED), jnp.float32),
        pltpu.SemaphoreType.DMA,
        pltpu.SemaphoreType.DMA,
    ],
    compiler_params=pltpu.CompilerParams(use_tc_tiling_on_sc=False),
)
def _gather_all(table_hbm, idx_hbm, out_hbm, idx_v, rows_v, sem_g, sem_s):
    wid = lax.axis_index("s") * NUM_CORES + lax.axis_index("c")
    base = wid * PER_W
    pltpu.sync_copy(idx_hbm.at[pl.ds(base, PER_W)], idx_v)

    def group(g, carry):
        goff = g * GROUP
        gat = [
            pltpu.async_copy(
                table_hbm.at[idx_v.at[pl.ds(goff + b * CHUNK, CHUNK)]],
                rows_v.at[b], sem_g)
            for b in range(NBUF)
        ]
        sto = []
        for b in range(NBUF):
            gat[b].wait()
            sto.append(pltpu.async_copy(
                rows_v.at[b],
                out_hbm.at[pl.ds(base + goff + b * CHUNK, CHUNK)], sem_s))
        for d in sto:
            d.wait()
        return carry

    lax.fori_loop(0, NGROUP, group, 0)


def kernel(inputs, labels, E):
    idx = jnp.concatenate(
        [inputs.reshape(-1), labels.reshape(-1)]).astype(jnp.int32)
    out = _gather_all(E, idx)
    return out.reshape(2, BATCH, WINDOW, EMBED)


# speedup vs baseline: 1.5913x; 1.4980x over previous
import functools

import jax
import jax.numpy as jnp
from jax import lax
from jax.experimental import pallas as pl
from jax.experimental.pallas import tpu as pltpu
from jax.experimental.pallas import tpu_sc as plsc

VOCAB = 1000000
EMBED = 64
BATCH = 4096
WINDOW = 200

TOTAL = 2 * BATCH * WINDOW
NUM_CORES = 2
NUM_SUBCORES = 16
NW = NUM_CORES * NUM_SUBCORES
PER_W = TOTAL // NW
CHUNK = 256
NBUF = 4
GROUP = NBUF * CHUNK
NGROUP = PER_W // GROUP

_mesh = plsc.VectorSubcoreMesh(
    core_axis_name="c", subcore_axis_name="s",
    num_cores=NUM_CORES, num_subcores=NUM_SUBCORES,
)


@functools.partial(
    pl.kernel,
    out_type=jax.ShapeDtypeStruct((TOTAL, 128), jnp.float32),
    mesh=_mesh,
    scratch_types=[
        pltpu.VMEM((PER_W,), jnp.int32),
        pltpu.VMEM((NBUF, CHUNK, EMBED), jnp.float32),
        pltpu.SemaphoreType.DMA,
        pltpu.SemaphoreType.DMA,
    ],
    compiler_params=pltpu.CompilerParams(use_tc_tiling_on_sc=False),
)
def _gather_all(table_hbm, idx_hbm, out_hbm, idx_v, rows_v, sem_g, sem_s):
    wid = lax.axis_index("s") * NUM_CORES + lax.axis_index("c")
    base = wid * PER_W
    pltpu.sync_copy(idx_hbm.at[pl.ds(base, PER_W)], idx_v)

    def group(g, carry):
        goff = g * GROUP
        gat = [
            pltpu.async_copy(
                table_hbm.at[idx_v.at[pl.ds(goff + b * CHUNK, CHUNK)]],
                rows_v.at[b], sem_g)
            for b in range(NBUF)
        ]
        sto = []
        for b in range(NBUF):
            gat[b].wait()
            sto.append(pltpu.async_copy(
                rows_v.at[b],
                out_hbm.at[pl.ds(base + goff + b * CHUNK, CHUNK),
                           pl.ds(0, EMBED)], sem_s))
        for d in sto:
            d.wait()
        return carry

    lax.fori_loop(0, NGROUP, group, 0)


def kernel(inputs, labels, E):
    idx = jnp.concatenate(
        [inputs.reshape(-1), labels.reshape(-1)]).astype(jnp.int32)
    out = _gather_all(E, idx)
    return out[:, :EMBED].reshape(2, BATCH, WINDOW, EMBED)
